# baseline (speedup 1.0000x reference)
import functools

import jax
import jax.numpy as jnp
from jax import lax
from jax.experimental import pallas as pl
from jax.experimental.pallas import tpu as pltpu
from jax.experimental.pallas import tpu_sc as plsc

VOCAB = 1000000
DIM = 64
ROWS = 4096
COLS = 200
NUM_WORKERS = 32
ROWS_PER_W = ROWS // NUM_WORKERS
IDX_PER_W = ROWS_PER_W * COLS
SUB0 = 128
SUB1 = COLS - SUB0
NBUF = 8
NROUNDS = ROWS_PER_W // NBUF

_mesh = plsc.VectorSubcoreMesh(core_axis_name="c", subcore_axis_name="s")


@functools.partial(
    pl.kernel,
    out_type=jax.ShapeDtypeStruct((ROWS, COLS, DIM), jnp.float32),
    mesh=_mesh,
    scratch_types=[
        pltpu.VMEM((IDX_PER_W,), jnp.int32),
        pltpu.VMEM((NBUF, COLS, DIM), jnp.float32),
        pltpu.SemaphoreType.DMA,
        pltpu.SemaphoreType.DMA((NBUF,)),
        pltpu.SemaphoreType.DMA((NBUF,)),
    ],
    compiler_params=pltpu.CompilerParams(use_tc_tiling_on_sc=False),
)
def _embed_gather(x_hbm, table_hbm, out_hbm, idx_v, rows_v, isem, gsem, wsem):
    wid = lax.axis_index("s") * 2 + lax.axis_index("c")
    row0 = pl.multiple_of(wid * ROWS_PER_W, ROWS_PER_W)

    def fire_gather(b, r):
        off = pl.multiple_of(r * COLS, 8)
        pltpu.async_copy(
            table_hbm.at[idx_v.at[pl.ds(off, SUB0)]],
            rows_v.at[b, pl.ds(0, SUB0), :],
            gsem.at[b],
        )
        off1 = pl.multiple_of(r * COLS + SUB0, 8)
        pltpu.async_copy(
            table_hbm.at[idx_v.at[pl.ds(off1, SUB1)]],
            rows_v.at[b, pl.ds(SUB0, SUB1), :],
            gsem.at[b],
        )

    def wait_gather(b, r):
        off = pl.multiple_of(r * COLS, 8)
        pltpu.make_async_copy(
            table_hbm.at[idx_v.at[pl.ds(off, SUB0)]],
            rows_v.at[b, pl.ds(0, SUB0), :],
            gsem.at[b],
        ).wait()
        off1 = pl.multiple_of(r * COLS + SUB0, 8)
        pltpu.make_async_copy(
            table_hbm.at[idx_v.at[pl.ds(off1, SUB1)]],
            rows_v.at[b, pl.ds(SUB0, SUB1), :],
            gsem.at[b],
        ).wait()

    def fire_writeback(b, r):
        pltpu.async_copy(rows_v.at[b], out_hbm.at[row0 + r], wsem.at[b])

    def wait_writeback(b, r):
        pltpu.make_async_copy(
            rows_v.at[b], out_hbm.at[row0 + r], wsem.at[b]
        ).wait()

    pltpu.async_copy(
        x_hbm.at[pl.ds(pl.multiple_of(wid * IDX_PER_W, 8), IDX_PER_W)],
        idx_v,
        isem,
    ).wait()

    for b in range(NBUF):
        fire_gather(b, b)

    def round_body(g, carry):
        r0 = g * NBUF
        for b in range(NBUF):
            wait_gather(b, r0 + b)
            fire_writeback(b, r0 + b)
        for b in range(NBUF):
            wait_writeback(b, r0 + b)
            fire_gather(b, r0 + b + NBUF)
        return carry

    lax.fori_loop(0, NROUNDS - 1, round_body, 0, unroll=False)

    r0 = (NROUNDS - 1) * NBUF
    for b in range(NBUF):
        wait_gather(b, r0 + b)
        fire_writeback(b, r0 + b)
    for b in range(NBUF):
        wait_writeback(b, r0 + b)


def kernel(x, table):
    flat_x = x.reshape(-1).astype(jnp.int32)
    return _embed_gather(flat_x, table)
